# baseline (speedup 1.0000x reference)
import functools

import jax
import jax.numpy as jnp
from jax import lax
from jax.experimental import pallas as pl
from jax.experimental.pallas import tpu as pltpu
from jax.experimental.pallas import tpu_sc as plsc

N_LANES = 16
N_WORKERS = 32
CHUNK = 128
STRIDE = 17


def _row_product(p_ref, q_ref, r, rank):
    terms = []
    for c in range(rank // N_LANES):
        p = p_ref[r, pl.ds(c * N_LANES, N_LANES)]
        q = q_ref[r, pl.ds(c * N_LANES, N_LANES)]
        terms.append(1.0 + jnp.exp(-(p + q)))
    while len(terms) > 1:
        terms = [a * b for a, b in zip(terms[::2], terms[1::2])]
    return terms[0]


GROUP_WORDS = N_LANES * STRIDE


def _compute_chunk(p_ref, q_ref, out_ref, stage, out_base, rank):
    lane = lax.iota(jnp.int32, N_LANES)
    zero = jnp.zeros((N_LANES,), jnp.int32)
    col_idx = lane * STRIDE

    @plsc.parallel_loop(0, CHUNK // N_LANES)
    def group_body(g):
        gbase = g * GROUP_WORDS

        @plsc.parallel_loop(0, N_LANES, step=2)
        def row_body(r):
            for rr in range(2):
                m = _row_product(p_ref, q_ref, g * N_LANES + r + rr, rank)
                stage[0, pl.ds(gbase + (r + rr) * STRIDE, N_LANES)] = m

        gcol = col_idx + gbase
        acc = plsc.load_gather(stage, [zero, gcol])
        for l in range(1, N_LANES):
            acc = acc * plsc.load_gather(stage, [zero, gcol + l])
        out_ref[pl.ds(out_base + g * N_LANES, N_LANES)] = 1.0 / acc


N_BUF = 3


def _sc_kernel(rows_per_w, i_hbm, j_hbm, p_hbm, q_hbm, out_hbm,
               iv, jv, pb0, pb1, pb2, qb0, qb1, qb2, outv, stage,
               isem, sem0, sem1, sem2):
    nchunks = rows_per_w // CHUNK
    wid = lax.axis_index("s") * 2 + lax.axis_index("c")
    base = wid * rows_per_w

    idx_copies = []
    for c in range(nchunks):
        idx_copies.append(
            pltpu.async_copy(i_hbm.at[pl.ds(base + c * CHUNK, CHUNK)], iv.at[c], isem))
        idx_copies.append(
            pltpu.async_copy(j_hbm.at[pl.ds(base + c * CHUNK, CHUNK)], jv.at[c], isem))
    for d in idx_copies:
        d.wait()

    pbufs, qbufs, sems = (pb0, pb1, pb2), (qb0, qb1, qb2), (sem0, sem1, sem2)

    def issue(c):
        s = c % N_BUF
        return (pltpu.async_copy(p_hbm.at[iv.at[c]], pbufs[s], sems[s]),
                pltpu.async_copy(q_hbm.at[jv.at[c]], qbufs[s], sems[s]))

    pending = {c: issue(c) for c in range(min(N_BUF, nchunks))}
    for c in range(nchunks):
        for d in pending.pop(c):
            d.wait()
        s = c % N_BUF
        _compute_chunk(pbufs[s], qbufs[s], outv, stage, c * CHUNK,
                       pbufs[s].shape[1])
        if c + N_BUF < nchunks:
            pending[c + N_BUF] = issue(c + N_BUF)

    pltpu.sync_copy(outv, out_hbm.at[pl.ds(base, rows_per_w)])


def kernel(i, j, P, Q):
    batch = i.shape[0]
    rows_per_w = batch // N_WORKERS
    nchunks = rows_per_w // CHUNK
    rank = P.shape[1]

    mesh = plsc.VectorSubcoreMesh(core_axis_name="c", subcore_axis_name="s")
    run = pl.kernel(
        functools.partial(_sc_kernel, rows_per_w),
        out_type=jax.ShapeDtypeStruct((batch,), jnp.float32),
        mesh=mesh,
        compiler_params=pltpu.CompilerParams(needs_layout_passes=False, disable_bounds_checks=True, disable_semaphore_checks=True, skip_device_barrier=True),
        scratch_types=[
            pltpu.VMEM((nchunks, CHUNK), jnp.int32),
            pltpu.VMEM((nchunks, CHUNK), jnp.int32),
            pltpu.VMEM((CHUNK, rank), jnp.float32),
            pltpu.VMEM((CHUNK, rank), jnp.float32),
            pltpu.VMEM((CHUNK, rank), jnp.float32),
            pltpu.VMEM((CHUNK, rank), jnp.float32),
            pltpu.VMEM((CHUNK, rank), jnp.float32),
            pltpu.VMEM((CHUNK, rank), jnp.float32),
            pltpu.VMEM((rows_per_w,), jnp.float32),
            pltpu.VMEM((1, (CHUNK // N_LANES) * N_LANES * STRIDE), jnp.float32),
            pltpu.SemaphoreType.DMA,
            pltpu.SemaphoreType.DMA,
            pltpu.SemaphoreType.DMA,
            pltpu.SemaphoreType.DMA,
        ],
    )
    out = run(i.astype(jnp.int32), j.astype(jnp.int32), P, Q)
    return out.reshape(-1, 1)


# speedup vs baseline: 1.0145x; 1.0145x over previous
import functools

import jax
import jax.numpy as jnp
from jax import lax
from jax.experimental import pallas as pl
from jax.experimental.pallas import tpu as pltpu
from jax.experimental.pallas import tpu_sc as plsc

N_LANES = 16
N_WORKERS = 32
CHUNK = 128
STRIDE = 17


def _row_product(p_ref, q_ref, r, rank):
    terms = []
    for c in range(rank // N_LANES):
        p = p_ref[r, pl.ds(c * N_LANES, N_LANES)]
        q = q_ref[r, pl.ds(c * N_LANES, N_LANES)]
        terms.append(1.0 + jnp.exp(-(p + q)))
    while len(terms) > 1:
        terms = [a * b for a, b in zip(terms[::2], terms[1::2])]
    return terms[0]


GROUP_WORDS = N_LANES * STRIDE


def _compute_chunk(p_ref, q_ref, out_ref, stage, out_base, rank):
    lane = lax.iota(jnp.int32, N_LANES)
    zero = jnp.zeros((N_LANES,), jnp.int32)
    col_idx = lane * STRIDE

    @plsc.parallel_loop(0, CHUNK // N_LANES)
    def group_body(g):
        gbase = g * GROUP_WORDS

        @plsc.parallel_loop(0, N_LANES, unroll=4)
        def row_body(r):
            m = _row_product(p_ref, q_ref, g * N_LANES + r, rank)
            stage[0, pl.ds(gbase + r * STRIDE, N_LANES)] = m

        gcol = col_idx + gbase
        acc = plsc.load_gather(stage, [zero, gcol])
        for l in range(1, N_LANES):
            acc = acc * plsc.load_gather(stage, [zero, gcol + l])
        out_ref[pl.ds(out_base + g * N_LANES, N_LANES)] = 1.0 / acc


N_BUF = 3


def _sc_kernel(rows_per_w, i_hbm, j_hbm, p_hbm, q_hbm, out_hbm,
               iv, jv, pb0, pb1, pb2, qb0, qb1, qb2, outv, stage,
               isem, sem0, sem1, sem2):
    nchunks = rows_per_w // CHUNK
    wid = lax.axis_index("s") * 2 + lax.axis_index("c")
    base = wid * rows_per_w

    idx_copies = []
    for c in range(nchunks):
        idx_copies.append(
            pltpu.async_copy(i_hbm.at[pl.ds(base + c * CHUNK, CHUNK)], iv.at[c], isem))
        idx_copies.append(
            pltpu.async_copy(j_hbm.at[pl.ds(base + c * CHUNK, CHUNK)], jv.at[c], isem))
    for d in idx_copies:
        d.wait()

    pbufs, qbufs, sems = (pb0, pb1, pb2), (qb0, qb1, qb2), (sem0, sem1, sem2)

    def issue(c):
        s = c % N_BUF
        return (pltpu.async_copy(p_hbm.at[iv.at[c]], pbufs[s], sems[s]),
                pltpu.async_copy(q_hbm.at[jv.at[c]], qbufs[s], sems[s]))

    pending = {c: issue(c) for c in range(min(N_BUF, nchunks))}
    for c in range(nchunks):
        for d in pending.pop(c):
            d.wait()
        s = c % N_BUF
        _compute_chunk(pbufs[s], qbufs[s], outv, stage, c * CHUNK,
                       pbufs[s].shape[1])
        if c + N_BUF < nchunks:
            pending[c + N_BUF] = issue(c + N_BUF)

    pltpu.sync_copy(outv, out_hbm.at[pl.ds(base, rows_per_w)])


def kernel(i, j, P, Q):
    batch = i.shape[0]
    rows_per_w = batch // N_WORKERS
    nchunks = rows_per_w // CHUNK
    rank = P.shape[1]

    mesh = plsc.VectorSubcoreMesh(core_axis_name="c", subcore_axis_name="s")
    run = pl.kernel(
        functools.partial(_sc_kernel, rows_per_w),
        out_type=jax.ShapeDtypeStruct((batch,), jnp.float32),
        mesh=mesh,
        compiler_params=pltpu.CompilerParams(needs_layout_passes=False),
        scratch_types=[
            pltpu.VMEM((nchunks, CHUNK), jnp.int32),
            pltpu.VMEM((nchunks, CHUNK), jnp.int32),
            pltpu.VMEM((CHUNK, rank), jnp.float32),
            pltpu.VMEM((CHUNK, rank), jnp.float32),
            pltpu.VMEM((CHUNK, rank), jnp.float32),
            pltpu.VMEM((CHUNK, rank), jnp.float32),
            pltpu.VMEM((CHUNK, rank), jnp.float32),
            pltpu.VMEM((CHUNK, rank), jnp.float32),
            pltpu.VMEM((rows_per_w,), jnp.float32),
            pltpu.VMEM((1, (CHUNK // N_LANES) * N_LANES * STRIDE), jnp.float32),
            pltpu.SemaphoreType.DMA,
            pltpu.SemaphoreType.DMA,
            pltpu.SemaphoreType.DMA,
            pltpu.SemaphoreType.DMA,
        ],
    )
    out = run(i.astype(jnp.int32), j.astype(jnp.int32), P, Q)
    return out.reshape(-1, 1)
